# baseline (speedup 1.0000x reference)
import functools

import jax
import jax.numpy as jnp
from jax import lax
from jax.experimental import pallas as pl
from jax.experimental.pallas import tpu as pltpu

B, T, C = 16, 2048, 512
BT = 256
NT = T // BT


def _tc_body(len_ref, x_ref, o_ref):
    b = pl.program_id(0)
    i = pl.program_id(1)
    l = len_ref[b]
    t0 = i * BT
    rows = lax.broadcasted_iota(jnp.int32, (BT, C), 0) + t0
    vals = jnp.where(rows < l, x_ref[0], jnp.float32(-10000.0))
    bmax = jnp.max(vals, axis=0, keepdims=True)[None]

    @pl.when(i == 0)
    def _():
        o_ref[...] = bmax

    @pl.when(i > 0)
    def _():
        o_ref[...] = jnp.maximum(o_ref[...], bmax)

    @pl.when(i == NT - 1)
    def _():
        o_ref[...] = jnp.where(l > 0, o_ref[...], jnp.float32(0.0))


@jax.jit
def tc_masked_max(features, lengths):
    grid_spec = pltpu.PrefetchScalarGridSpec(
        num_scalar_prefetch=1,
        grid=(B, NT),
        in_specs=[pl.BlockSpec((1, BT, C), lambda b, i, lens: (b, i, 0))],
        out_specs=pl.BlockSpec((1, 1, C), lambda b, i, lens: (b, 0, 0)),
    )
    return pl.pallas_call(
        _tc_body,
        grid_spec=grid_spec,
        out_shape=jax.ShapeDtypeStruct((B, 1, C), jnp.float32),
    )(lengths.astype(jnp.int32), features)[:, 0, :]


def kernel(features, lengths):
    return tc_masked_max(features, lengths)


# speedup vs baseline: 1.6088x; 1.6088x over previous
import functools

import jax
import jax.numpy as jnp
from jax import lax
from jax.experimental import pallas as pl
from jax.experimental.pallas import tpu as pltpu
from jax.experimental.pallas import tpu_sc as plsc

B, T, C = 16, 2048, 512
BSC = 10
BTC = B - BSC
L = 16
NC, NS = 2, 16
CH = C // NC
NGH = CH // L
R = 64
BT = 256
NT = T // BT

_NEG = float("-inf")

_mesh = plsc.VectorSubcoreMesh(core_axis_name="c", subcore_axis_name="s")


@functools.partial(
    pl.kernel,
    mesh=_mesh,
    out_type=jax.ShapeDtypeStruct((BSC, C), jnp.float32),
    scratch_types=[
        pltpu.VMEM((L,), jnp.int32),
        pltpu.VMEM((R, CH), jnp.float32),
        pltpu.VMEM((R, CH), jnp.float32),
        pltpu.VMEM((BSC, CH), jnp.float32),
        pltpu.VMEM((NS, CH), jnp.float32),
        pltpu.VMEM_SHARED((NS, BSC, CH), jnp.float32),
        pltpu.SemaphoreType.DMA,
        pltpu.SemaphoreType.DMA,
    ],
)
def _sc_masked_max(feat_hbm, len_hbm, out_hbm, len_v, buf0, buf1, accv, mrg_v,
                   shared, sem0, sem1):
    c = lax.axis_index("c")
    s = lax.axis_index("s")
    c0 = c * CH

    pltpu.sync_copy(len_hbm, len_v)
    lvec = len_v[...]
    lens = [jnp.clip(lvec[b], 0, T) for b in range(BSC)]

    pref = [jnp.int32(0)]
    for b in range(BSC):
        pref.append(pref[b] + (lens[b] + R - 1) // R)
    ntot = pref[BSC]

    def decode(t):
        bat = jnp.int32(0)
        base = jnp.int32(0)
        blen = lens[0]
        for b in range(1, BSC):
            here = t >= pref[b]
            bat = jnp.where(here, b, bat)
            base = jnp.where(here, pref[b], base)
            blen = jnp.where(here, lens[b], blen)
        t0 = (t - base) * R
        nrows = jnp.maximum(jnp.minimum(blen - t0, R), 0)
        return bat, t0, nrows

    nitems = jnp.maximum((ntot - s + NS - 1) // NS, 0)
    bufs = (buf0, buf1)
    sems = (sem0, sem1)

    def initb(b, carry):
        neg = jnp.full((L,), _NEG, jnp.float32)
        for g in range(NGH):
            accv[b, pl.ds(g * L, L)] = neg
        return carry

    lax.fori_loop(0, BSC, initb, jnp.int32(0))

    def start_copy(k, kbuf):
        bat, t0, _ = decode(s + k * NS)
        pltpu.make_async_copy(
            feat_hbm.at[bat, pl.ds(t0, R), pl.ds(c0, CH)],
            bufs[kbuf], sems[kbuf]).start()

    @pl.when(nitems > 0)
    def _():
        start_copy(0, 0)

    @pl.when(nitems > 1)
    def _():
        start_copy(1, 1)

    def step(k, kbuf):
        @pl.when(k < nitems)
        def _():
            pltpu.make_async_copy(
                feat_hbm.at[0, pl.ds(0, R), pl.ds(c0, CH)],
                bufs[kbuf], sems[kbuf]).wait()

        @pl.when(k + 2 < nitems)
        def _():
            start_copy(k + 2, kbuf)

        bat, _, nrows = decode(s + k * NS)
        buf = bufs[kbuf]
        acc = tuple(accv[bat, pl.ds(g * L, L)] for g in range(NGH))

        def row2_body(r, acc):
            return tuple(
                jnp.maximum(acc[g], jnp.maximum(buf[2 * r, pl.ds(g * L, L)],
                                                buf[2 * r + 1, pl.ds(g * L, L)]))
                for g in range(NGH)
            )

        acc = lax.fori_loop(0, nrows // 2, row2_body, acc)

        odd = (nrows % 2) == 1
        last = jnp.maximum(nrows - 1, 0)
        acc = tuple(
            jnp.where(odd, jnp.maximum(acc[g], buf[last, pl.ds(g * L, L)]),
                      acc[g])
            for g in range(NGH)
        )
        for g in range(NGH):
            accv[bat, pl.ds(g * L, L)] = acc[g]

    def pair_body(j, carry):
        step(2 * j, 0)
        step(2 * j + 1, 1)
        return carry

    lax.fori_loop(0, (nitems + 1) // 2, pair_body, jnp.int32(0))

    pltpu.sync_copy(accv, shared.at[s])
    plsc.subcore_barrier()
    sm = jnp.minimum(s, BSC - 1)
    pltpu.sync_copy(shared.at[:, sm, :], mrg_v)

    mylen = jnp.int32(0)
    for b in range(BSC):
        mylen = jnp.where(sm == b, lens[b], mylen)
    nonzero = mylen > 0
    full = mylen >= T

    def mrg_body(r, v):
        return tuple(
            jnp.maximum(v[g], mrg_v[r, pl.ds(g * L, L)]) for g in range(NGH))

    v0 = tuple(mrg_v[0, pl.ds(g * L, L)] for g in range(NGH))
    vm = lax.fori_loop(1, NS, mrg_body, v0)
    for g in range(NGH):
        v = jnp.where(full, vm[g], jnp.maximum(vm[g], jnp.float32(-10000.0)))
        mrg_v[0, pl.ds(g * L, L)] = jnp.where(nonzero, v, jnp.float32(0.0))

    @pl.when(s < BSC)
    def _():
        pltpu.sync_copy(mrg_v.at[0], out_hbm.at[s, pl.ds(c0, CH)])


def _tc_body(len_ref, x_ref, o_ref):
    i = pl.program_id(1)
    l = len_ref[BSC + pl.program_id(0)]
    t0 = i * BT

    @pl.when(i == 0)
    def _():
        o_ref[...] = jnp.full((1, 1, C), -10000.0, jnp.float32)

    @pl.when(t0 + BT <= l)
    def _():
        o_ref[...] = jnp.maximum(
            o_ref[...], jnp.max(x_ref[0], axis=0, keepdims=True)[None])

    @pl.when((t0 < l) & (t0 + BT > l))
    def _():
        rows = lax.broadcasted_iota(jnp.int32, (BT, C), 0) + t0
        vals = jnp.where(rows < l, x_ref[0], jnp.float32(-10000.0))
        o_ref[...] = jnp.maximum(
            o_ref[...], jnp.max(vals, axis=0, keepdims=True)[None])

    @pl.when((i == NT - 1) & (l <= 0))
    def _():
        o_ref[...] = jnp.zeros((1, 1, C), jnp.float32)


def _tc_masked_max(features, lengths):
    grid_spec = pltpu.PrefetchScalarGridSpec(
        num_scalar_prefetch=1,
        grid=(BTC, NT),
        in_specs=[pl.BlockSpec((1, BT, C), lambda b, i, lens: (BSC + b, i, 0))],
        out_specs=pl.BlockSpec((1, 1, C), lambda b, i, lens: (b, 0, 0)),
    )
    return pl.pallas_call(
        _tc_body,
        grid_spec=grid_spec,
        out_shape=jax.ShapeDtypeStruct((BTC, 1, C), jnp.float32),
    )(lengths, features)[:, 0, :]


def kernel(features, lengths):
    lens = lengths.astype(jnp.int32)
    sc_out = _sc_masked_max(features, lens)
    tc_out = _tc_masked_max(features, lens)
    return jnp.concatenate([sc_out, tc_out], axis=0)


# speedup vs baseline: 1.6195x; 1.0066x over previous
import functools

import jax
import jax.numpy as jnp
from jax import lax
from jax.experimental import pallas as pl
from jax.experimental.pallas import tpu as pltpu
from jax.experimental.pallas import tpu_sc as plsc

B, T, C = 16, 2048, 512
BSC = 10
BTC = B - BSC
L = 16
NC, NS = 2, 16
CH = C // NC
NGH = CH // L
R = 64
BT = 256
NT = T // BT

_NEG = float("-inf")

_mesh = plsc.VectorSubcoreMesh(core_axis_name="c", subcore_axis_name="s")


@functools.partial(
    pl.kernel,
    mesh=_mesh,
    out_type=jax.ShapeDtypeStruct((BSC, C), jnp.float32),
    scratch_types=[
        pltpu.VMEM((L,), jnp.int32),
        pltpu.VMEM((R, CH), jnp.float32),
        pltpu.VMEM((R, CH), jnp.float32),
        pltpu.VMEM((BSC, CH), jnp.float32),
        pltpu.VMEM((NS, CH), jnp.float32),
        pltpu.VMEM_SHARED((NS, BSC, CH), jnp.float32),
        pltpu.SemaphoreType.DMA,
        pltpu.SemaphoreType.DMA,
    ],
)
def _sc_masked_max(feat_hbm, len_hbm, out_hbm, len_v, buf0, buf1, accv, mrg_v,
                   shared, sem0, sem1):
    c = lax.axis_index("c")
    s = lax.axis_index("s")
    c0 = c * CH

    pltpu.sync_copy(len_hbm, len_v)
    lvec = len_v[...]
    lens = [jnp.clip(lvec[b], 0, T) for b in range(BSC)]

    pref = [jnp.int32(0)]
    for b in range(BSC):
        pref.append(pref[b] + (lens[b] + R - 1) // R)
    ntot = pref[BSC]

    def decode(t):
        bat = jnp.int32(0)
        base = jnp.int32(0)
        blen = lens[0]
        for b in range(1, BSC):
            here = t >= pref[b]
            bat = jnp.where(here, b, bat)
            base = jnp.where(here, pref[b], base)
            blen = jnp.where(here, lens[b], blen)
        t0 = (t - base) * R
        nrows = jnp.maximum(jnp.minimum(blen - t0, R), 0)
        return bat, t0, nrows

    nitems = jnp.maximum((ntot - s + NS - 1) // NS, 0)
    bufs = (buf0, buf1)
    sems = (sem0, sem1)

    def initb(b, carry):
        neg = jnp.full((L,), _NEG, jnp.float32)
        for g in range(NGH):
            accv[b, pl.ds(g * L, L)] = neg
        return carry

    lax.fori_loop(0, BSC, initb, jnp.int32(0))

    def start_copy(k, kbuf):
        bat, t0, _ = decode(s + k * NS)
        pltpu.make_async_copy(
            feat_hbm.at[bat, pl.ds(t0, R), pl.ds(c0, CH)],
            bufs[kbuf], sems[kbuf]).start()

    @pl.when(nitems > 0)
    def _():
        start_copy(0, 0)

    @pl.when(nitems > 1)
    def _():
        start_copy(1, 1)

    def step(k, kbuf):
        @pl.when(k < nitems)
        def _():
            pltpu.make_async_copy(
                feat_hbm.at[0, pl.ds(0, R), pl.ds(c0, CH)],
                bufs[kbuf], sems[kbuf]).wait()

        bat, _, nrows = decode(s + k * NS)
        buf = bufs[kbuf]
        acc = tuple(accv[bat, pl.ds(g * L, L)] for g in range(NGH))

        def row2_body(r, acc):
            return tuple(
                jnp.maximum(acc[g], jnp.maximum(buf[2 * r, pl.ds(g * L, L)],
                                                buf[2 * r + 1, pl.ds(g * L, L)]))
                for g in range(NGH)
            )

        acc = lax.fori_loop(0, nrows // 2, row2_body, acc)

        odd = (nrows % 2) == 1
        last = jnp.maximum(nrows - 1, 0)
        acc = tuple(
            jnp.where(odd, jnp.maximum(acc[g], buf[last, pl.ds(g * L, L)]),
                      acc[g])
            for g in range(NGH)
        )
        for g in range(NGH):
            accv[bat, pl.ds(g * L, L)] = acc[g]

        @pl.when(k + 2 < nitems)
        def _():
            start_copy(k + 2, kbuf)

    def pair_body(j, carry):
        step(2 * j, 0)
        step(2 * j + 1, 1)
        return carry

    lax.fori_loop(0, (nitems + 1) // 2, pair_body, jnp.int32(0))

    pltpu.sync_copy(accv, shared.at[s])
    plsc.subcore_barrier()
    sm = jnp.minimum(s, BSC - 1)
    pltpu.sync_copy(shared.at[:, sm, :], mrg_v)

    mylen = jnp.int32(0)
    for b in range(BSC):
        mylen = jnp.where(sm == b, lens[b], mylen)
    nonzero = mylen > 0
    full = mylen >= T

    def mrg_body(r, v):
        return tuple(
            jnp.maximum(v[g], mrg_v[r, pl.ds(g * L, L)]) for g in range(NGH))

    v0 = tuple(mrg_v[0, pl.ds(g * L, L)] for g in range(NGH))
    vm = lax.fori_loop(1, NS, mrg_body, v0)
    for g in range(NGH):
        v = jnp.where(full, vm[g], jnp.maximum(vm[g], jnp.float32(-10000.0)))
        mrg_v[0, pl.ds(g * L, L)] = jnp.where(nonzero, v, jnp.float32(0.0))

    @pl.when(s < BSC)
    def _():
        pltpu.sync_copy(mrg_v.at[0], out_hbm.at[s, pl.ds(c0, CH)])


def _tc_body(len_ref, x_ref, o_ref):
    i = pl.program_id(1)
    l = len_ref[BSC + pl.program_id(0)]
    t0 = i * BT

    @pl.when(i == 0)
    def _():
        o_ref[...] = jnp.full((1, 1, C), -10000.0, jnp.float32)

    @pl.when(t0 + BT <= l)
    def _():
        o_ref[...] = jnp.maximum(
            o_ref[...], jnp.max(x_ref[0], axis=0, keepdims=True)[None])

    @pl.when((t0 < l) & (t0 + BT > l))
    def _():
        rows = lax.broadcasted_iota(jnp.int32, (BT, C), 0) + t0
        vals = jnp.where(rows < l, x_ref[0], jnp.float32(-10000.0))
        o_ref[...] = jnp.maximum(
            o_ref[...], jnp.max(vals, axis=0, keepdims=True)[None])

    @pl.when((i == NT - 1) & (l <= 0))
    def _():
        o_ref[...] = jnp.zeros((1, 1, C), jnp.float32)


def _tc_masked_max(features, lengths):
    grid_spec = pltpu.PrefetchScalarGridSpec(
        num_scalar_prefetch=1,
        grid=(BTC, NT),
        in_specs=[pl.BlockSpec((1, BT, C), lambda b, i, lens: (BSC + b, i, 0))],
        out_specs=pl.BlockSpec((1, 1, C), lambda b, i, lens: (b, 0, 0)),
    )
    return pl.pallas_call(
        _tc_body,
        grid_spec=grid_spec,
        out_shape=jax.ShapeDtypeStruct((BTC, 1, C), jnp.float32),
    )(lengths, features)[:, 0, :]


def kernel(features, lengths):
    lens = lengths.astype(jnp.int32)
    sc_out = _sc_masked_max(features, lens)
    tc_out = _tc_masked_max(features, lens)
    return jnp.concatenate([sc_out, tc_out], axis=0)


# speedup vs baseline: 1.6304x; 1.0068x over previous
import functools

import jax
import jax.numpy as jnp
from jax import lax
from jax.experimental import pallas as pl
from jax.experimental.pallas import tpu as pltpu
from jax.experimental.pallas import tpu_sc as plsc

B, T, C = 16, 2048, 512
BSC = 10
BTC = B - BSC
L = 16
NC, NS = 2, 16
CH = C // NC
NGH = CH // L
R = 64
BT = 256
NT = T // BT

_NEG = float("-inf")

_mesh = plsc.VectorSubcoreMesh(core_axis_name="c", subcore_axis_name="s")


@functools.partial(
    pl.kernel,
    mesh=_mesh,
    out_type=jax.ShapeDtypeStruct((BSC, C), jnp.float32),
    scratch_types=[
        pltpu.VMEM((L,), jnp.int32),
        pltpu.VMEM((R, CH), jnp.float32),
        pltpu.VMEM((R, CH), jnp.float32),
        pltpu.VMEM((NS, CH), jnp.float32),
        pltpu.VMEM((NS, CH), jnp.float32),
        pltpu.VMEM_SHARED((NS, NS, CH), jnp.float32),
        pltpu.SemaphoreType.DMA,
        pltpu.SemaphoreType.DMA,
    ],
)
def _sc_masked_max(feat_hbm, len_hbm, out_hbm, len_v, buf0, buf1, accv, mrg_v,
                   shared, sem0, sem1):
    c = lax.axis_index("c")
    s = lax.axis_index("s")
    c0 = c * CH

    pltpu.sync_copy(len_hbm, len_v)
    lvec = len_v[...]
    lens = [jnp.clip(lvec[b], 0, T) for b in range(BSC)]

    pref = [jnp.int32(0)]
    for b in range(BSC):
        pref.append(pref[b] + (lens[b] + R - 1) // R)
    ntot = pref[BSC]

    def decode(t):
        bat = jnp.int32(0)
        base = jnp.int32(0)
        blen = lens[0]
        for b in range(1, BSC):
            here = t >= pref[b]
            bat = jnp.where(here, b, bat)
            base = jnp.where(here, pref[b], base)
            blen = jnp.where(here, lens[b], blen)
        t0 = (t - base) * R
        nrows = jnp.maximum(jnp.minimum(blen - t0, R), 0)
        return bat, t0, nrows

    nitems = jnp.maximum((ntot - s + NS - 1) // NS, 0)
    bufs = (buf0, buf1)
    sems = (sem0, sem1)

    def initb(b, carry):
        neg = jnp.full((L,), _NEG, jnp.float32)
        for g in range(NGH):
            accv[b, pl.ds(g * L, L)] = neg
        return carry

    lax.fori_loop(0, NS, initb, jnp.int32(0))

    def start_copy(k, kbuf):
        bat, t0, _ = decode(s + k * NS)
        pltpu.make_async_copy(
            feat_hbm.at[bat, pl.ds(t0, R), pl.ds(c0, CH)],
            bufs[kbuf], sems[kbuf]).start()

    @pl.when(nitems > 0)
    def _():
        start_copy(0, 0)

    @pl.when(nitems > 1)
    def _():
        start_copy(1, 1)

    def step(k, kbuf):
        @pl.when(k < nitems)
        def _():
            pltpu.make_async_copy(
                feat_hbm.at[0, pl.ds(0, R), pl.ds(c0, CH)],
                bufs[kbuf], sems[kbuf]).wait()

        bat, _, nrows = decode(s + k * NS)
        buf = bufs[kbuf]
        acc = tuple(accv[bat, pl.ds(g * L, L)] for g in range(NGH))

        def row2_body(r, acc):
            return tuple(
                jnp.maximum(acc[g], jnp.maximum(buf[2 * r, pl.ds(g * L, L)],
                                                buf[2 * r + 1, pl.ds(g * L, L)]))
                for g in range(NGH)
            )

        acc = lax.fori_loop(0, nrows // 2, row2_body, acc)

        odd = (nrows % 2) == 1
        last = jnp.maximum(nrows - 1, 0)
        acc = tuple(
            jnp.where(odd, jnp.maximum(acc[g], buf[last, pl.ds(g * L, L)]),
                      acc[g])
            for g in range(NGH)
        )
        for g in range(NGH):
            accv[bat, pl.ds(g * L, L)] = acc[g]

        @pl.when(k + 2 < nitems)
        def _():
            start_copy(k + 2, kbuf)

    def pair_body(j, carry):
        step(2 * j, 0)
        step(2 * j + 1, 1)
        return carry

    lax.fori_loop(0, (nitems + 1) // 2, pair_body, jnp.int32(0))

    pltpu.sync_copy(accv, shared.at[s])
    plsc.subcore_barrier()
    sm = jnp.minimum(s, BSC - 1)
    pltpu.sync_copy(shared.at[:, sm, :], mrg_v)

    mylen = jnp.int32(0)
    for b in range(BSC):
        mylen = jnp.where(sm == b, lens[b], mylen)
    nonzero = mylen > 0
    full = mylen >= T

    def mrg_body(r, v):
        return tuple(
            jnp.maximum(v[g], mrg_v[r, pl.ds(g * L, L)]) for g in range(NGH))

    v0 = tuple(mrg_v[0, pl.ds(g * L, L)] for g in range(NGH))
    vm = lax.fori_loop(1, NS, mrg_body, v0)
    for g in range(NGH):
        v = jnp.where(full, vm[g], jnp.maximum(vm[g], jnp.float32(-10000.0)))
        mrg_v[0, pl.ds(g * L, L)] = jnp.where(nonzero, v, jnp.float32(0.0))

    @pl.when(s < BSC)
    def _():
        pltpu.sync_copy(mrg_v.at[0], out_hbm.at[s, pl.ds(c0, CH)])


def _tc_body(len_ref, x_ref, o_ref):
    i = pl.program_id(1)
    l = len_ref[BSC + pl.program_id(0)]
    t0 = i * BT

    @pl.when(i == 0)
    def _():
        o_ref[...] = jnp.full((1, 1, C), -10000.0, jnp.float32)

    @pl.when(t0 + BT <= l)
    def _():
        o_ref[...] = jnp.maximum(
            o_ref[...], jnp.max(x_ref[0], axis=0, keepdims=True)[None])

    @pl.when((t0 < l) & (t0 + BT > l))
    def _():
        rows = lax.broadcasted_iota(jnp.int32, (BT, C), 0) + t0
        vals = jnp.where(rows < l, x_ref[0], jnp.float32(-10000.0))
        o_ref[...] = jnp.maximum(
            o_ref[...], jnp.max(vals, axis=0, keepdims=True)[None])

    @pl.when((i == NT - 1) & (l <= 0))
    def _():
        o_ref[...] = jnp.zeros((1, 1, C), jnp.float32)


def _tc_masked_max(features, lengths):
    grid_spec = pltpu.PrefetchScalarGridSpec(
        num_scalar_prefetch=1,
        grid=(BTC, NT),
        in_specs=[pl.BlockSpec((1, BT, C), lambda b, i, lens: (BSC + b, i, 0))],
        out_specs=pl.BlockSpec((1, 1, C), lambda b, i, lens: (b, 0, 0)),
    )
    return pl.pallas_call(
        _tc_body,
        grid_spec=grid_spec,
        out_shape=jax.ShapeDtypeStruct((BTC, 1, C), jnp.float32),
    )(lengths, features)[:, 0, :]


def kernel(features, lengths):
    lens = lengths.astype(jnp.int32)
    sc_out = _sc_masked_max(features, lens)
    tc_out = _tc_masked_max(features, lens)
    return jnp.concatenate([sc_out, tc_out], axis=0)


# speedup vs baseline: 1.9686x; 1.2074x over previous
import functools

import jax
import jax.numpy as jnp
from jax import lax
from jax.experimental import pallas as pl
from jax.experimental.pallas import tpu as pltpu
from jax.experimental.pallas import tpu_sc as plsc

B, T, C = 16, 2048, 512
BSC = 16
BTC = B - BSC
L = 16
NC, NS = 2, 16
CH = C // NC
NGH = CH // L
R = 64
BT = 256
NT = T // BT

_NEG = float("-inf")

_mesh = plsc.VectorSubcoreMesh(core_axis_name="c", subcore_axis_name="s")


@functools.partial(
    pl.kernel,
    mesh=_mesh,
    out_type=jax.ShapeDtypeStruct((BSC, C), jnp.float32),
    scratch_types=[
        pltpu.VMEM((L,), jnp.int32),
        pltpu.VMEM((R, CH), jnp.float32),
        pltpu.VMEM((R, CH), jnp.float32),
        pltpu.VMEM((NS, CH), jnp.float32),
        pltpu.VMEM((NS, CH), jnp.float32),
        pltpu.VMEM_SHARED((NS, NS, CH), jnp.float32),
        pltpu.SemaphoreType.DMA,
        pltpu.SemaphoreType.DMA,
    ],
)
def _sc_masked_max(feat_hbm, len_hbm, out_hbm, len_v, buf0, buf1, accv, mrg_v,
                   shared, sem0, sem1):
    c = lax.axis_index("c")
    s = lax.axis_index("s")
    c0 = c * CH

    pltpu.sync_copy(len_hbm, len_v)
    lvec = len_v[...]
    lens = [jnp.clip(lvec[b], 0, T) for b in range(BSC)]

    pref = [jnp.int32(0)]
    for b in range(BSC):
        pref.append(pref[b] + (lens[b] + R - 1) // R)
    ntot = pref[BSC]

    def decode(t):
        bat = jnp.int32(0)
        base = jnp.int32(0)
        blen = lens[0]
        for b in range(1, BSC):
            here = t >= pref[b]
            bat = jnp.where(here, b, bat)
            base = jnp.where(here, pref[b], base)
            blen = jnp.where(here, lens[b], blen)
        t0 = (t - base) * R
        nrows = jnp.maximum(jnp.minimum(blen - t0, R), 0)
        return bat, t0, nrows

    nitems = jnp.maximum((ntot - s + NS - 1) // NS, 0)
    bufs = (buf0, buf1)
    sems = (sem0, sem1)

    def initb(b, carry):
        neg = jnp.full((L,), _NEG, jnp.float32)
        for g in range(NGH):
            accv[b, pl.ds(g * L, L)] = neg
        return carry

    lax.fori_loop(0, NS, initb, jnp.int32(0))

    def start_copy(k, kbuf):
        bat, t0, _ = decode(s + k * NS)
        pltpu.make_async_copy(
            feat_hbm.at[bat, pl.ds(t0, R), pl.ds(c0, CH)],
            bufs[kbuf], sems[kbuf]).start()

    @pl.when(nitems > 0)
    def _():
        start_copy(0, 0)

    @pl.when(nitems > 1)
    def _():
        start_copy(1, 1)

    def step(k, kbuf):
        @pl.when(k < nitems)
        def _():
            pltpu.make_async_copy(
                feat_hbm.at[0, pl.ds(0, R), pl.ds(c0, CH)],
                bufs[kbuf], sems[kbuf]).wait()

        bat, _, nrows = decode(s + k * NS)
        buf = bufs[kbuf]
        acc = tuple(accv[bat, pl.ds(g * L, L)] for g in range(NGH))

        def row2_body(r, acc):
            return tuple(
                jnp.maximum(acc[g], jnp.maximum(buf[2 * r, pl.ds(g * L, L)],
                                                buf[2 * r + 1, pl.ds(g * L, L)]))
                for g in range(NGH)
            )

        acc = lax.fori_loop(0, nrows // 2, row2_body, acc)

        odd = (nrows % 2) == 1
        last = jnp.maximum(nrows - 1, 0)
        acc = tuple(
            jnp.where(odd, jnp.maximum(acc[g], buf[last, pl.ds(g * L, L)]),
                      acc[g])
            for g in range(NGH)
        )
        for g in range(NGH):
            accv[bat, pl.ds(g * L, L)] = acc[g]

        @pl.when(k + 2 < nitems)
        def _():
            start_copy(k + 2, kbuf)

    def pair_body(j, carry):
        step(2 * j, 0)
        step(2 * j + 1, 1)
        return carry

    lax.fori_loop(0, (nitems + 1) // 2, pair_body, jnp.int32(0))

    pltpu.sync_copy(accv, shared.at[s])
    plsc.subcore_barrier()
    sm = jnp.minimum(s, BSC - 1)
    pltpu.sync_copy(shared.at[:, sm, :], mrg_v)

    mylen = jnp.int32(0)
    for b in range(BSC):
        mylen = jnp.where(sm == b, lens[b], mylen)
    nonzero = mylen > 0
    full = mylen >= T

    def mrg_body(r, v):
        return tuple(
            jnp.maximum(v[g], mrg_v[r, pl.ds(g * L, L)]) for g in range(NGH))

    v0 = tuple(mrg_v[0, pl.ds(g * L, L)] for g in range(NGH))
    vm = lax.fori_loop(1, NS, mrg_body, v0)
    for g in range(NGH):
        v = jnp.where(full, vm[g], jnp.maximum(vm[g], jnp.float32(-10000.0)))
        mrg_v[0, pl.ds(g * L, L)] = jnp.where(nonzero, v, jnp.float32(0.0))

    @pl.when(s < BSC)
    def _():
        pltpu.sync_copy(mrg_v.at[0], out_hbm.at[s, pl.ds(c0, CH)])


def _tc_body(len_ref, x_ref, o_ref):
    i = pl.program_id(1)
    l = len_ref[BSC + pl.program_id(0)]
    t0 = i * BT

    @pl.when(i == 0)
    def _():
        o_ref[...] = jnp.full((1, 1, C), -10000.0, jnp.float32)

    @pl.when(t0 + BT <= l)
    def _():
        o_ref[...] = jnp.maximum(
            o_ref[...], jnp.max(x_ref[0], axis=0, keepdims=True)[None])

    @pl.when((t0 < l) & (t0 + BT > l))
    def _():
        rows = lax.broadcasted_iota(jnp.int32, (BT, C), 0) + t0
        vals = jnp.where(rows < l, x_ref[0], jnp.float32(-10000.0))
        o_ref[...] = jnp.maximum(
            o_ref[...], jnp.max(vals, axis=0, keepdims=True)[None])

    @pl.when((i == NT - 1) & (l <= 0))
    def _():
        o_ref[...] = jnp.zeros((1, 1, C), jnp.float32)


def _tc_masked_max(features, lengths):
    grid_spec = pltpu.PrefetchScalarGridSpec(
        num_scalar_prefetch=1,
        grid=(BTC, NT),
        in_specs=[pl.BlockSpec((1, BT, C), lambda b, i, lens: (BSC + b, i, 0))],
        out_specs=pl.BlockSpec((1, 1, C), lambda b, i, lens: (b, 0, 0)),
    )
    return pl.pallas_call(
        _tc_body,
        grid_spec=grid_spec,
        out_shape=jax.ShapeDtypeStruct((BTC, 1, C), jnp.float32),
    )(lengths, features)[:, 0, :]


def kernel(features, lengths):
    lens = lengths.astype(jnp.int32)
    sc_out = _sc_masked_max(features, lens)
    if BTC == 0:
        return sc_out
    tc_out = _tc_masked_max(features, lens)
    return jnp.concatenate([sc_out, tc_out], axis=0)


# speedup vs baseline: 2.1533x; 1.0939x over previous
import functools

import jax
import jax.numpy as jnp
from jax import lax
from jax.experimental import pallas as pl
from jax.experimental.pallas import tpu as pltpu
from jax.experimental.pallas import tpu_sc as plsc

B, T, C = 16, 2048, 512
BSC = 16
BTC = B - BSC
L = 16
NC, NS = 2, 16
CH = C // NC
NGH = CH // L
R = 64
BT = 256
NT = T // BT

_NEG = float("-inf")

_mesh = plsc.VectorSubcoreMesh(core_axis_name="c", subcore_axis_name="s")


@functools.partial(
    pl.kernel,
    mesh=_mesh,
    out_type=jax.ShapeDtypeStruct((BSC, C), jnp.float32),
    scratch_types=[
        pltpu.VMEM((L,), jnp.int32),
        pltpu.VMEM((R, CH), jnp.float32),
        pltpu.VMEM((R, CH), jnp.float32),
        pltpu.VMEM((R, CH), jnp.float32),
        pltpu.VMEM((NS, CH), jnp.float32),
        pltpu.VMEM((NS, CH), jnp.float32),
        pltpu.VMEM_SHARED((NS, NS, CH), jnp.float32),
        pltpu.SemaphoreType.DMA,
        pltpu.SemaphoreType.DMA,
        pltpu.SemaphoreType.DMA,
    ],
)
def _sc_masked_max(feat_hbm, len_hbm, out_hbm, len_v, buf0, buf1, buf2, accv,
                   mrg_v, shared, sem0, sem1, sem2):
    c = lax.axis_index("c")
    s = lax.axis_index("s")
    c0 = c * CH

    pltpu.sync_copy(len_hbm, len_v)
    lvec = len_v[...]
    lens = [jnp.clip(lvec[b], 0, T) for b in range(BSC)]

    pref = [jnp.int32(0)]
    for b in range(BSC):
        pref.append(pref[b] + (lens[b] + R - 1) // R)
    ntot = pref[BSC]

    def decode(t):
        bat = jnp.int32(0)
        base = jnp.int32(0)
        blen = lens[0]
        for b in range(1, BSC):
            here = t >= pref[b]
            bat = jnp.where(here, b, bat)
            base = jnp.where(here, pref[b], base)
            blen = jnp.where(here, lens[b], blen)
        t0 = (t - base) * R
        nrows = jnp.maximum(jnp.minimum(blen - t0, R), 0)
        return bat, t0, nrows

    nitems = jnp.maximum((ntot - s + NS - 1) // NS, 0)
    bufs = (buf0, buf1, buf2)
    sems = (sem0, sem1, sem2)

    def initb(b, carry):
        neg = jnp.full((L,), _NEG, jnp.float32)
        for g in range(NGH):
            accv[b, pl.ds(g * L, L)] = neg
        return carry

    lax.fori_loop(0, NS, initb, jnp.int32(0))

    def start_copy(k, kbuf):
        bat, t0, _ = decode(s + k * NS)
        pltpu.make_async_copy(
            feat_hbm.at[bat, pl.ds(t0, R), pl.ds(c0, CH)],
            bufs[kbuf], sems[kbuf]).start()

    @pl.when(nitems > 0)
    def _():
        start_copy(0, 0)

    @pl.when(nitems > 1)
    def _():
        start_copy(1, 1)

    def step(k, kbuf):
        @pl.when(k < nitems)
        def _():
            pltpu.make_async_copy(
                feat_hbm.at[0, pl.ds(0, R), pl.ds(c0, CH)],
                bufs[kbuf], sems[kbuf]).wait()

        @pl.when(k + 2 < nitems)
        def _():
            start_copy(k + 2, (kbuf + 2) % 3)

        bat, _, nrows = decode(s + k * NS)
        buf = bufs[kbuf]
        acc = tuple(accv[bat, pl.ds(g * L, L)] for g in range(NGH))

        def row2_body(r, acc):
            return tuple(
                jnp.maximum(acc[g], jnp.maximum(buf[2 * r, pl.ds(g * L, L)],
                                                buf[2 * r + 1, pl.ds(g * L, L)]))
                for g in range(NGH)
            )

        acc = lax.fori_loop(0, nrows // 2, row2_body, acc)

        odd = (nrows % 2) == 1
        last = jnp.maximum(nrows - 1, 0)
        acc = tuple(
            jnp.where(odd, jnp.maximum(acc[g], buf[last, pl.ds(g * L, L)]),
                      acc[g])
            for g in range(NGH)
        )
        for g in range(NGH):
            accv[bat, pl.ds(g * L, L)] = acc[g]

    def trio_body(j, carry):
        step(3 * j, 0)
        step(3 * j + 1, 1)
        step(3 * j + 2, 2)
        return carry

    lax.fori_loop(0, (nitems + 2) // 3, trio_body, jnp.int32(0))

    pltpu.sync_copy(accv, shared.at[s])
    plsc.subcore_barrier()
    sm = jnp.minimum(s, BSC - 1)
    pltpu.sync_copy(shared.at[:, sm, :], mrg_v)

    mylen = jnp.int32(0)
    for b in range(BSC):
        mylen = jnp.where(sm == b, lens[b], mylen)
    nonzero = mylen > 0
    full = mylen >= T

    def mrg_body(r, v):
        return tuple(
            jnp.maximum(v[g], mrg_v[r, pl.ds(g * L, L)]) for g in range(NGH))

    v0 = tuple(mrg_v[0, pl.ds(g * L, L)] for g in range(NGH))
    vm = lax.fori_loop(1, NS, mrg_body, v0)
    for g in range(NGH):
        v = jnp.where(full, vm[g], jnp.maximum(vm[g], jnp.float32(-10000.0)))
        mrg_v[0, pl.ds(g * L, L)] = jnp.where(nonzero, v, jnp.float32(0.0))

    @pl.when(s < BSC)
    def _():
        pltpu.sync_copy(mrg_v.at[0], out_hbm.at[s, pl.ds(c0, CH)])


def _tc_body(len_ref, x_ref, o_ref):
    i = pl.program_id(1)
    l = len_ref[BSC + pl.program_id(0)]
    t0 = i * BT

    @pl.when(i == 0)
    def _():
        o_ref[...] = jnp.full((1, 1, C), -10000.0, jnp.float32)

    @pl.when(t0 + BT <= l)
    def _():
        o_ref[...] = jnp.maximum(
            o_ref[...], jnp.max(x_ref[0], axis=0, keepdims=True)[None])

    @pl.when((t0 < l) & (t0 + BT > l))
    def _():
        rows = lax.broadcasted_iota(jnp.int32, (BT, C), 0) + t0
        vals = jnp.where(rows < l, x_ref[0], jnp.float32(-10000.0))
        o_ref[...] = jnp.maximum(
            o_ref[...], jnp.max(vals, axis=0, keepdims=True)[None])

    @pl.when((i == NT - 1) & (l <= 0))
    def _():
        o_ref[...] = jnp.zeros((1, 1, C), jnp.float32)


def _tc_masked_max(features, lengths):
    grid_spec = pltpu.PrefetchScalarGridSpec(
        num_scalar_prefetch=1,
        grid=(BTC, NT),
        in_specs=[pl.BlockSpec((1, BT, C), lambda b, i, lens: (BSC + b, i, 0))],
        out_specs=pl.BlockSpec((1, 1, C), lambda b, i, lens: (b, 0, 0)),
    )
    return pl.pallas_call(
        _tc_body,
        grid_spec=grid_spec,
        out_shape=jax.ShapeDtypeStruct((BTC, 1, C), jnp.float32),
    )(lengths, features)[:, 0, :]


def kernel(features, lengths):
    lens = lengths.astype(jnp.int32)
    sc_out = _sc_masked_max(features, lens)
    if BTC == 0:
        return sc_out
    tc_out = _tc_masked_max(features, lens)
    return jnp.concatenate([sc_out, tc_out], axis=0)
